# baseline (speedup 1.0000x reference)
import jax
import jax.numpy as jnp
from jax import lax
from jax.experimental import pallas as pl
from jax.experimental.pallas import tpu as pltpu
from jax.experimental.pallas import tpu_sc as plsc

N = 64
M = 1024
NC = 2
NS = 16
LANES = 16
NW = NC * NS

B_ROWS = 16384
ROW = 200
COLS_PER_W = B_ROWS // NW
RB = 40
NCHUNK = ROW // RB
CVECS = COLS_PER_W // LANES


def _sc_body(pidx, midx, pt, mt, out,
             pt_v, mt_v,
             pidx_v0, pidx_v1, midx_v0, midx_v1, out_v0, out_v1,
             insem0, insem1, outsem0, outsem1, tsem, hsem):
    wid = lax.axis_index("s") * NC + lax.axis_index("c")
    col0 = wid * COLS_PER_W

    pbufs = (pidx_v0, pidx_v1)
    mbufs = (midx_v0, midx_v1)
    obufs = (out_v0, out_v1)
    insems = (insem0, insem1)
    outsems = (outsem0, outsem1)

    def start_in(c, b):
        r = c * RB
        src_p = pidx.at[pl.ds(r, RB), pl.ds(col0, COLS_PER_W)]
        src_m = midx.at[pl.ds(r, RB), pl.ds(col0, COLS_PER_W)]
        dp = pltpu.async_copy(src_p, pbufs[b], insems[b])
        dm = pltpu.async_copy(src_m, mbufs[b], insems[b])
        return dp, dm

    HEAD = 8

    def start_head():
        src_p = pidx.at[pl.ds(0, HEAD), pl.ds(col0, COLS_PER_W)]
        src_m = midx.at[pl.ds(0, HEAD), pl.ds(col0, COLS_PER_W)]
        dp = pltpu.async_copy(src_p, pbufs[0].at[pl.ds(0, HEAD), :], hsem)
        dm = pltpu.async_copy(src_m, mbufs[0].at[pl.ds(0, HEAD), :], hsem)
        return dp, dm

    def start_rest():
        src_p = pidx.at[pl.ds(HEAD, RB - HEAD), pl.ds(col0, COLS_PER_W)]
        src_m = midx.at[pl.ds(HEAD, RB - HEAD), pl.ds(col0, COLS_PER_W)]
        dp = pltpu.async_copy(src_p, pbufs[0].at[pl.ds(HEAD, RB - HEAD), :], insems[0])
        dm = pltpu.async_copy(src_m, mbufs[0].at[pl.ds(HEAD, RB - HEAD), :], insems[0])
        return dp, dm

    dt0 = pltpu.async_copy(pt, pt_v, tsem)
    dt1 = pltpu.async_copy(mt, mt_v, tsem)
    dhead = start_head()
    drest = start_rest()
    din = {1: start_in(1, 1)}
    dt0.wait()
    dt1.wait()

    def compute(pv, mv, ov, lo, hi):
        @plsc.parallel_loop(lo * CVECS, hi * CVECS, unroll=4)
        def _(i):
            r = lax.shift_right_logical(i, 5)
            s = pl.ds(lax.shift_left(lax.bitwise_and(i, 31), 4), LANES)
            piv = jnp.minimum(jnp.maximum(pv[r, s], 0), N - 1)
            miv = jnp.minimum(jnp.maximum(mv[r, s], 0), M - 1)
            ov[r, s] = plsc.load_gather(pt_v, [piv]) * plsc.load_gather(mt_v, [miv])

    dout = {}
    for c in range(NCHUNK):
        b = c % 2
        pv, mv, ov = pbufs[b], mbufs[b], obufs[b]

        if c == 0:
            for d in dhead:
                d.wait()
            compute(pv, mv, ov, 0, HEAD)
            for d in drest:
                d.wait()
            compute(pv, mv, ov, HEAD, RB)
        else:
            dp, dm = din.pop(c)
            dp.wait()
            dm.wait()
            if c >= 2:
                dout[b].wait()
            compute(pv, mv, ov, 0, RB)

        dout[b] = pltpu.async_copy(
            ov, out.at[pl.ds(c * RB, RB), pl.ds(col0, COLS_PER_W)], outsems[b])
        if c + 2 < NCHUNK:
            din[c + 2] = start_in(c + 2, b)

    dout[0].wait()
    dout[1].wait()


def kernel(phase_indices, mag_indices, phase_cos_table, mag_exp_sin_table):
    pidx = phase_indices.astype(jnp.int32).T
    midx = mag_indices.astype(jnp.int32).T
    pt = phase_cos_table.astype(jnp.float32)
    mt = mag_exp_sin_table.astype(jnp.float32)
    mesh = plsc.VectorSubcoreMesh(core_axis_name="c", subcore_axis_name="s")
    out = pl.kernel(
        _sc_body,
        mesh=mesh,
        compiler_params=pltpu.CompilerParams(needs_layout_passes=False),
        out_type=jax.ShapeDtypeStruct((ROW, B_ROWS), jnp.float32),
        scratch_types=[
            pltpu.VMEM((N,), jnp.float32),
            pltpu.VMEM((M,), jnp.float32),
            pltpu.VMEM((RB, COLS_PER_W), jnp.int32),
            pltpu.VMEM((RB, COLS_PER_W), jnp.int32),
            pltpu.VMEM((RB, COLS_PER_W), jnp.int32),
            pltpu.VMEM((RB, COLS_PER_W), jnp.int32),
            pltpu.VMEM((RB, COLS_PER_W), jnp.float32),
            pltpu.VMEM((RB, COLS_PER_W), jnp.float32),
            pltpu.SemaphoreType.DMA,
            pltpu.SemaphoreType.DMA,
            pltpu.SemaphoreType.DMA,
            pltpu.SemaphoreType.DMA,
            pltpu.SemaphoreType.DMA,
            pltpu.SemaphoreType.DMA,
        ],
    )(pidx, midx, pt, mt)
    return out.T


# speedup vs baseline: 1.0051x; 1.0051x over previous
import jax
import jax.numpy as jnp
from jax import lax
from jax.experimental import pallas as pl
from jax.experimental.pallas import tpu as pltpu
from jax.experimental.pallas import tpu_sc as plsc

N = 64
M = 1024
NC = 2
NS = 16
LANES = 16
NW = NC * NS

B_ROWS = 16384
ROW = 200
COLS_PER_W = B_ROWS // NW
RB = 24
HEAD = 8
CHUNKS = [(i * RB, RB) for i in range(8)] + [(8 * RB, HEAD)]
NCHUNK = len(CHUNKS)
CVECS = COLS_PER_W // LANES


def _sc_body(pidx, midx, pt, mt, out,
             pt_v, mt_v,
             pi0, pi1, pi2, mi0, mi1, mi2, ov0, ov1,
             insem0, insem1, insem2, outsem0, outsem1, tsem, hsem):
    wid = lax.axis_index("s") * NC + lax.axis_index("c")
    col0 = wid * COLS_PER_W

    pbufs = (pi0, pi1, pi2)
    mbufs = (mi0, mi1, mi2)
    obufs = (ov0, ov1)
    insems = (insem0, insem1, insem2)
    outsems = (outsem0, outsem1)

    def start_in(c, r0=0, sem=None):
        slot = c % 3
        base, nr = CHUNKS[c]
        s = insems[slot] if sem is None else sem
        src_p = pidx.at[pl.ds(base + r0, nr - r0), pl.ds(col0, COLS_PER_W)]
        src_m = midx.at[pl.ds(base + r0, nr - r0), pl.ds(col0, COLS_PER_W)]
        dp = pltpu.async_copy(src_p, pbufs[slot].at[pl.ds(r0, nr - r0), :], s)
        dm = pltpu.async_copy(src_m, mbufs[slot].at[pl.ds(r0, nr - r0), :], s)
        return dp, dm

    dt0 = pltpu.async_copy(pt, pt_v, tsem)
    dt1 = pltpu.async_copy(mt, mt_v, tsem)
    dhead = pltpu.async_copy(
        pidx.at[pl.ds(0, HEAD), pl.ds(col0, COLS_PER_W)],
        pbufs[0].at[pl.ds(0, HEAD), :], hsem)
    dhead2 = pltpu.async_copy(
        midx.at[pl.ds(0, HEAD), pl.ds(col0, COLS_PER_W)],
        mbufs[0].at[pl.ds(0, HEAD), :], hsem)
    drest = start_in(0, r0=HEAD)
    din = {1: start_in(1), 2: start_in(2)}
    dt0.wait()
    dt1.wait()

    def compute(pv, mv, ov, lo, hi):
        @plsc.parallel_loop(lo * CVECS, hi * CVECS, unroll=4)
        def _(i):
            r = lax.shift_right_logical(i, 5)
            s = pl.ds(lax.shift_left(lax.bitwise_and(i, 31), 4), LANES)
            piv = jnp.minimum(jnp.maximum(pv[r, s], 0), N - 1)
            miv = jnp.minimum(jnp.maximum(mv[r, s], 0), M - 1)
            ov[r, s] = plsc.load_gather(pt_v, [piv]) * plsc.load_gather(mt_v, [miv])

    dout = {}
    for c in range(NCHUNK):
        slot = c % 3
        b = c % 2
        base, nr = CHUNKS[c]
        pv, mv, ov = pbufs[slot], mbufs[slot], obufs[b]

        if c == 0:
            dhead.wait()
            dhead2.wait()
            compute(pv, mv, ov, 0, HEAD)
            for d in drest:
                d.wait()
            compute(pv, mv, ov, HEAD, RB)
        else:
            dp, dm = din.pop(c)
            dp.wait()
            dm.wait()
            if c >= 2:
                dout[b].wait()
            compute(pv, mv, ov, 0, nr)

        dout[b] = pltpu.async_copy(
            ov.at[pl.ds(0, nr), :],
            out.at[pl.ds(base, nr), pl.ds(col0, COLS_PER_W)], outsems[b])
        if c + 3 < NCHUNK:
            din[c + 3] = start_in(c + 3)

    dout[0].wait()
    dout[1].wait()


def kernel(phase_indices, mag_indices, phase_cos_table, mag_exp_sin_table):
    pidx = phase_indices.astype(jnp.int32).T
    midx = mag_indices.astype(jnp.int32).T
    pt = phase_cos_table.astype(jnp.float32)
    mt = mag_exp_sin_table.astype(jnp.float32)
    mesh = plsc.VectorSubcoreMesh(core_axis_name="c", subcore_axis_name="s")
    out = pl.kernel(
        _sc_body,
        mesh=mesh,
        compiler_params=pltpu.CompilerParams(needs_layout_passes=False),
        out_type=jax.ShapeDtypeStruct((ROW, B_ROWS), jnp.float32),
        scratch_types=[
            pltpu.VMEM((N,), jnp.float32),
            pltpu.VMEM((M,), jnp.float32),
            pltpu.VMEM((RB, COLS_PER_W), jnp.int32),
            pltpu.VMEM((RB, COLS_PER_W), jnp.int32),
            pltpu.VMEM((RB, COLS_PER_W), jnp.int32),
            pltpu.VMEM((RB, COLS_PER_W), jnp.int32),
            pltpu.VMEM((RB, COLS_PER_W), jnp.int32),
            pltpu.VMEM((RB, COLS_PER_W), jnp.int32),
            pltpu.VMEM((RB, COLS_PER_W), jnp.float32),
            pltpu.VMEM((RB, COLS_PER_W), jnp.float32),
            pltpu.SemaphoreType.DMA,
            pltpu.SemaphoreType.DMA,
            pltpu.SemaphoreType.DMA,
            pltpu.SemaphoreType.DMA,
            pltpu.SemaphoreType.DMA,
            pltpu.SemaphoreType.DMA,
            pltpu.SemaphoreType.DMA,
        ],
    )(pidx, midx, pt, mt)
    return out.T
